# Initial kernel scaffold; baseline (speedup 1.0000x reference)
import jax
import jax.numpy as jnp
from jax.experimental import pallas as pl


def kernel(x, edge_index, edge_weight, W):
    raise NotImplementedError("write your pallas kernel here")


# speedup vs baseline: 5.6389x; 5.6389x over previous
import dataclasses
import functools

import jax
import jax.numpy as jnp
from jax import lax
from jax.experimental import pallas as pl
from jax.experimental.pallas import tpu as pltpu
from jax.experimental.pallas import tpu_sc as plsc

N_NODES = 10000
D = 128
LANES = 16
NC = 2
NS = 16
NW = NC * NS
E_TOTAL = 320000
EPW = E_TOTAL // NW
B = 80
SEG = 2000
NSTG = EPW // SEG
BPS = SEG // B
N_PAD = 10240
RPT = N_PAD // NS


def _sc_spmm(x, src, dst_blocks, w):
    mesh = plsc.VectorSubcoreMesh(core_axis_name="c", subcore_axis_name="s")

    cp = pltpu.CompilerParams()
    if "needs_layout_passes" in pltpu.CompilerParams.__dataclass_fields__:
        cp = dataclasses.replace(cp, needs_layout_passes=False)

    @functools.partial(
        pl.kernel,
        compiler_params=cp,
        out_type=jax.ShapeDtypeStruct((NC, N_PAD, D), jnp.float32),
        mesh=mesh,
        scratch_types=[
            pltpu.VMEM((SEG,), jnp.int32),
            pltpu.VMEM((BPS, B), jnp.int32),
            pltpu.VMEM((SEG,), jnp.float32),
            pltpu.VMEM((B, D), jnp.float32),
            pltpu.VMEM_SHARED((N_PAD, D), jnp.float32),
        ],
    )
    def spmm(x_hbm, src_hbm, dst_hbm, w_hbm, out_hbm,
             src_v, dst_v, w_v, rows_v, accum):
        cid = lax.axis_index("c")
        sid = lax.axis_index("s")
        wid = cid * NS + sid
        base = wid * EPW

        @pl.loop(0, B)
        def _(r):
            for c in range(D // LANES):
                rows_v.at[r, pl.ds(c * LANES, LANES)][...] = jnp.zeros(
                    (LANES,), jnp.float32)

        for k in range(RPT // B):
            pltpu.sync_copy(rows_v, accum.at[pl.ds(sid * RPT + k * B, B)])
        plsc.subcore_barrier()

        @pl.loop(0, NSTG)
        def _(s):
            pltpu.sync_copy(src_hbm.at[pl.ds(base + s * SEG, SEG)], src_v)
            pltpu.sync_copy(dst_hbm.at[wid, s], dst_v)
            pltpu.sync_copy(w_hbm.at[pl.ds(base + s * SEG, SEG)], w_v)

            @pl.loop(0, BPS)
            def _(j):
                pltpu.sync_copy(x_hbm.at[src_v.at[pl.ds(j * B, B)]], rows_v)

                @pl.loop(0, B)
                def _(i):
                    wb = plsc.load_gather(
                        w_v, [jnp.full((LANES,), j * B + i, jnp.int32)])
                    for c in range(D // LANES):
                        sl = (i, pl.ds(c * LANES, LANES))
                        rows_v.at[sl[0], sl[1]][...] = (
                            rows_v.at[sl[0], sl[1]][...] * wb)

                pltpu.sync_copy(rows_v, accum.at[dst_v.at[j]], add=True)

        plsc.subcore_barrier()

        pltpu.sync_copy(accum.at[pl.ds(sid * RPT, RPT)],
                        out_hbm.at[cid, pl.ds(sid * RPT, RPT)])

    return spmm(x, src, dst_blocks, w)


def _tc_matmul_relu(p0, p1, W):
    BLK = 1000

    def mm(p0_ref, p1_ref, w_ref, o_ref):
        acc = p0_ref[...] + p1_ref[...]
        o_ref[...] = jnp.maximum(
            jnp.dot(acc, w_ref[...], preferred_element_type=jnp.float32), 0.0)

    return pl.pallas_call(
        mm,
        grid=(N_NODES // BLK,),
        in_specs=[
            pl.BlockSpec((BLK, D), lambda i: (i, 0)),
            pl.BlockSpec((BLK, D), lambda i: (i, 0)),
            pl.BlockSpec((D, D), lambda i: (0, 0)),
        ],
        out_specs=pl.BlockSpec((BLK, D), lambda i: (i, 0)),
        out_shape=jax.ShapeDtypeStruct((N_NODES, D), jnp.float32),
    )(p0, p1, W)


@jax.jit
def kernel(x, edge_index, edge_weight, W):
    src = edge_index[0]
    dst_blocks = edge_index[1].reshape(NW, NSTG, BPS, B)
    partials = _sc_spmm(x, src, dst_blocks, edge_weight)
    return _tc_matmul_relu(partials[0, :N_NODES], partials[1, :N_NODES],
                           W)


# speedup vs baseline: 7.4611x; 1.3231x over previous
import dataclasses
import functools

import jax
import jax.numpy as jnp
from jax import lax
from jax.experimental import pallas as pl
from jax.experimental.pallas import tpu as pltpu
from jax.experimental.pallas import tpu_sc as plsc

N_NODES = 10000
D = 128
LANES = 16
NC = 2
NS = 16
NW = NC * NS
E_TOTAL = 320000
EPW = E_TOTAL // NW
B = 40
SEG = 2000
NSTG = EPW // SEG
BPS = SEG // B
N_PAD = 10240
RPT = N_PAD // NS


def _sc_spmm(x, src, dst_blocks, w):
    mesh = plsc.VectorSubcoreMesh(core_axis_name="c", subcore_axis_name="s")

    cp = pltpu.CompilerParams()
    if "needs_layout_passes" in pltpu.CompilerParams.__dataclass_fields__:
        cp = dataclasses.replace(cp, needs_layout_passes=False)

    @functools.partial(
        pl.kernel,
        compiler_params=cp,
        out_type=jax.ShapeDtypeStruct((NC, N_PAD, D), jnp.float32),
        mesh=mesh,
        scratch_types=[
            pltpu.VMEM((SEG,), jnp.int32),
            pltpu.VMEM((BPS, B), jnp.int32),
            pltpu.VMEM((SEG,), jnp.float32),
            pltpu.VMEM((B, D), jnp.float32),
            pltpu.VMEM((B, D), jnp.float32),
            pltpu.VMEM_SHARED((N_PAD, D), jnp.float32),
            pltpu.SemaphoreType.DMA,
            pltpu.SemaphoreType.DMA,
        ],
    )
    def spmm(x_hbm, src_hbm, dst_hbm, w_hbm, out_hbm,
             src_v, dst_v, w_v, rows0, rows1, accum, gs0, gs1):
        cid = lax.axis_index("c")
        sid = lax.axis_index("s")
        wid = cid * NS + sid
        base = wid * EPW

        @pl.loop(0, B)
        def _(r):
            for c in range(D // LANES):
                rows0.at[r, pl.ds(c * LANES, LANES)][...] = jnp.zeros(
                    (LANES,), jnp.float32)

        for k in range(RPT // B):
            pltpu.sync_copy(rows0, accum.at[pl.ds(sid * RPT + k * B, B)])
        plsc.subcore_barrier()

        def start_gather(j, rows, sem):
            off = jnp.minimum(j, BPS - 1) * B
            pltpu.async_copy(x_hbm.at[src_v.at[pl.ds(off, B)]], rows, sem)

        def wait_gather(rows, sem):
            pltpu.make_async_copy(x_hbm.at[pl.ds(0, B)], rows, sem).wait()

        def scale(j, rows):
            @pl.loop(0, B)
            def _(i):
                wb = plsc.load_gather(
                    w_v, [jnp.full((LANES,), j * B + i, jnp.int32)])
                for c in range(D // LANES):
                    rows.at[i, pl.ds(c * LANES, LANES)][...] = (
                        rows.at[i, pl.ds(c * LANES, LANES)][...] * wb)

        @pl.loop(0, NSTG)
        def _(s):
            pltpu.sync_copy(src_hbm.at[pl.ds(base + s * SEG, SEG)], src_v)
            pltpu.sync_copy(dst_hbm.at[wid, s], dst_v)
            pltpu.sync_copy(w_hbm.at[pl.ds(base + s * SEG, SEG)], w_v)

            start_gather(0, rows0, gs0)
            start_gather(1, rows1, gs1)

            @pl.loop(0, BPS // 2)
            def _(k):
                j0 = 2 * k
                wait_gather(rows0, gs0)
                scale(j0, rows0)
                pltpu.sync_copy(rows0, accum.at[dst_v.at[j0]], add=True)
                start_gather(j0 + 2, rows0, gs0)

                wait_gather(rows1, gs1)
                scale(j0 + 1, rows1)
                pltpu.sync_copy(rows1, accum.at[dst_v.at[j0 + 1]], add=True)
                start_gather(j0 + 3, rows1, gs1)

            wait_gather(rows0, gs0)
            wait_gather(rows1, gs1)

        plsc.subcore_barrier()

        pltpu.sync_copy(accum.at[pl.ds(sid * RPT, RPT)],
                        out_hbm.at[cid, pl.ds(sid * RPT, RPT)])

    return spmm(x, src, dst_blocks, w)


def _tc_matmul_relu(p0, p1, W):
    BLK = 1000

    def mm(p0_ref, p1_ref, w_ref, o_ref):
        acc = p0_ref[...] + p1_ref[...]
        o_ref[...] = jnp.maximum(
            jnp.dot(acc, w_ref[...], preferred_element_type=jnp.float32), 0.0)

    return pl.pallas_call(
        mm,
        grid=(N_NODES // BLK,),
        in_specs=[
            pl.BlockSpec((BLK, D), lambda i: (i, 0)),
            pl.BlockSpec((BLK, D), lambda i: (i, 0)),
            pl.BlockSpec((D, D), lambda i: (0, 0)),
        ],
        out_specs=pl.BlockSpec((BLK, D), lambda i: (i, 0)),
        out_shape=jax.ShapeDtypeStruct((N_NODES, D), jnp.float32),
    )(p0, p1, W)


@jax.jit
def kernel(x, edge_index, edge_weight, W):
    src = edge_index[0]
    dst_blocks = edge_index[1].reshape(NW, NSTG, BPS, B)
    partials = _sc_spmm(x, src, dst_blocks, edge_weight)
    return _tc_matmul_relu(partials[0, :N_NODES], partials[1, :N_NODES],
                           W)


# speedup vs baseline: 10.3992x; 1.3938x over previous
import dataclasses
import functools

import jax
import jax.numpy as jnp
from jax import lax
from jax.experimental import pallas as pl
from jax.experimental.pallas import tpu as pltpu
from jax.experimental.pallas import tpu_sc as plsc

N_NODES = 10000
D = 128
LANES = 16
NC = 2
NS = 16
NW = NC * NS
E_TOTAL = 320000
EPW = E_TOTAL // NW
B = 40
SEG = 2000
NSTG = EPW // SEG
BPS = SEG // B
NBUF = 4
GROUPS = (BPS - 2) // NBUF
TAIL = BPS - GROUPS * NBUF
N_PAD = 10240
RPT = N_PAD // NS


def _sc_spmm(x, src, dst_blocks, w):
    mesh = plsc.VectorSubcoreMesh(core_axis_name="c", subcore_axis_name="s")

    cp = pltpu.CompilerParams()
    if "needs_layout_passes" in pltpu.CompilerParams.__dataclass_fields__:
        cp = dataclasses.replace(cp, needs_layout_passes=False)

    @functools.partial(
        pl.kernel,
        compiler_params=cp,
        out_type=jax.ShapeDtypeStruct((NC, N_PAD, D), jnp.float32),
        mesh=mesh,
        scratch_types=[
            pltpu.VMEM((SEG,), jnp.int32),
            pltpu.VMEM((BPS, B), jnp.int32),
            pltpu.VMEM((SEG,), jnp.float32),
            *[pltpu.VMEM((B, D), jnp.float32)] * NBUF,
            pltpu.VMEM_SHARED((N_PAD, D), jnp.float32),
            *[pltpu.SemaphoreType.DMA] * NBUF,
            *[pltpu.SemaphoreType.DMA] * NBUF,
        ],
    )
    def spmm(x_hbm, src_hbm, dst_hbm, w_hbm, out_hbm,
             src_v, dst_v, w_v, r0, r1, r2, r3, accum,
             g0, g1, g2, g3, s0, s1, s2, s3):
        rows = [r0, r1, r2, r3]
        gs = [g0, g1, g2, g3]
        ss = [s0, s1, s2, s3]
        cid = lax.axis_index("c")
        sid = lax.axis_index("s")
        wid = cid * NS + sid
        base = wid * EPW

        @pl.loop(0, B)
        def _(r):
            for c in range(D // LANES):
                rows[0].at[r, pl.ds(c * LANES, LANES)][...] = jnp.zeros(
                    (LANES,), jnp.float32)

        for k in range(RPT // B):
            pltpu.sync_copy(rows[0], accum.at[pl.ds(sid * RPT + k * B, B)])
        plsc.subcore_barrier()

        def start_gather(j, b):
            off = jnp.minimum(j, BPS - 1) * B
            pltpu.async_copy(x_hbm.at[src_v.at[pl.ds(off, B)]], rows[b],
                             gs[b])

        def wait_gather(b):
            pltpu.make_async_copy(x_hbm.at[pl.ds(0, B)], rows[b],
                                  gs[b]).wait()

        def start_scatter(j, b):
            pltpu.async_copy(rows[b], accum.at[dst_v.at[j]], ss[b],
                             add=True)

        def wait_scatter(b):
            pltpu.make_async_copy(rows[b], accum.at[pl.ds(0, B)],
                                  ss[b]).wait()

        def scale(j, b):
            @pl.loop(0, B, unroll=4)
            def _(i):
                wb = plsc.load_gather(
                    w_v, [jnp.full((LANES,), j * B + i, jnp.int32)])
                for c in range(D // LANES):
                    rows[b].at[i, pl.ds(c * LANES, LANES)][...] = (
                        rows[b].at[i, pl.ds(c * LANES, LANES)][...] * wb)

        @pl.loop(0, NSTG)
        def _(s):
            pltpu.sync_copy(src_hbm.at[pl.ds(base + s * SEG, SEG)], src_v)
            pltpu.sync_copy(dst_hbm.at[wid, s], dst_v)
            pltpu.sync_copy(w_hbm.at[pl.ds(base + s * SEG, SEG)], w_v)

            for b in range(NBUF):
                start_gather(b, b)

            @pl.loop(0, GROUPS)
            def _(k):
                j = NBUF * k
                for b in range(NBUF):
                    wait_gather(b)
                    scale(j + b, b)
                    start_scatter(j + b, b)
                    if b >= 1:
                        wait_scatter(b - 1)
                        start_gather(j + NBUF + b - 1, b - 1)
                wait_scatter(NBUF - 1)
                start_gather(j + 2 * NBUF - 1, NBUF - 1)

            for t in range(TAIL):
                wait_gather(t)
                scale(GROUPS * NBUF + t, t)
                start_scatter(GROUPS * NBUF + t, t)
            for t in range(TAIL):
                wait_scatter(t)
            for b in range(TAIL, NBUF):
                wait_gather(b)

        plsc.subcore_barrier()

        pltpu.sync_copy(accum.at[pl.ds(sid * RPT, RPT)],
                        out_hbm.at[cid, pl.ds(sid * RPT, RPT)])

    return spmm(x, src, dst_blocks, w)


def _tc_matmul_relu(p0, p1, W):
    BLK = 1000

    def mm(p0_ref, p1_ref, w_ref, o_ref):
        acc = p0_ref[...] + p1_ref[...]
        o_ref[...] = jnp.maximum(
            jnp.dot(acc, w_ref[...], preferred_element_type=jnp.float32), 0.0)

    return pl.pallas_call(
        mm,
        grid=(N_NODES // BLK,),
        in_specs=[
            pl.BlockSpec((BLK, D), lambda i: (i, 0)),
            pl.BlockSpec((BLK, D), lambda i: (i, 0)),
            pl.BlockSpec((D, D), lambda i: (0, 0)),
        ],
        out_specs=pl.BlockSpec((BLK, D), lambda i: (i, 0)),
        out_shape=jax.ShapeDtypeStruct((N_NODES, D), jnp.float32),
    )(p0, p1, W)


@jax.jit
def kernel(x, edge_index, edge_weight, W):
    src = edge_index[0]
    dst_blocks = edge_index[1].reshape(NW, NSTG, BPS, B)
    partials = _sc_spmm(x, src, dst_blocks, edge_weight)
    return _tc_matmul_relu(partials[0, :N_NODES], partials[1, :N_NODES],
                           W)


# speedup vs baseline: 10.7052x; 1.0294x over previous
import dataclasses
import functools

import jax
import jax.numpy as jnp
from jax import lax
from jax.experimental import pallas as pl
from jax.experimental.pallas import tpu as pltpu
from jax.experimental.pallas import tpu_sc as plsc

N_NODES = 10000
D = 128
LANES = 16
NC = 2
NS = 16
NW = NC * NS
E_TOTAL = 320000
EPW = E_TOTAL // NW
B = 40
SEG = 2000
NSTG = EPW // SEG
BPS = SEG // B
NBUF = 6
GROUPS = (BPS - 2) // NBUF
TAIL = BPS - GROUPS * NBUF
N_PAD = 10240
RPT = N_PAD // NS


def _sc_spmm(x, src, dst_blocks, w):
    mesh = plsc.VectorSubcoreMesh(core_axis_name="c", subcore_axis_name="s")

    cp = pltpu.CompilerParams()
    if "needs_layout_passes" in pltpu.CompilerParams.__dataclass_fields__:
        cp = dataclasses.replace(cp, needs_layout_passes=False)

    @functools.partial(
        pl.kernel,
        compiler_params=cp,
        out_type=jax.ShapeDtypeStruct((NC, N_PAD, D), jnp.float32),
        mesh=mesh,
        scratch_types=[
            pltpu.VMEM((SEG,), jnp.int32),
            pltpu.VMEM((BPS, B), jnp.int32),
            pltpu.VMEM((SEG,), jnp.float32),
            *[pltpu.VMEM((B, D), jnp.float32)] * NBUF,
            pltpu.VMEM_SHARED((N_PAD, D), jnp.float32),
            *[pltpu.SemaphoreType.DMA] * NBUF,
            *[pltpu.SemaphoreType.DMA] * NBUF,
        ],
    )
    def spmm(x_hbm, src_hbm, dst_hbm, w_hbm, out_hbm,
             src_v, dst_v, w_v, r0, r1, r2, r3, r4, r5, accum,
             g0, g1, g2, g3, g4, g5, s0, s1, s2, s3, s4, s5):
        rows = [r0, r1, r2, r3, r4, r5]
        gs = [g0, g1, g2, g3, g4, g5]
        ss = [s0, s1, s2, s3, s4, s5]
        cid = lax.axis_index("c")
        sid = lax.axis_index("s")
        wid = cid * NS + sid
        base = wid * EPW

        @pl.loop(0, B)
        def _(r):
            for c in range(D // LANES):
                rows[0].at[r, pl.ds(c * LANES, LANES)][...] = jnp.zeros(
                    (LANES,), jnp.float32)

        for k in range(RPT // B):
            pltpu.sync_copy(rows[0], accum.at[pl.ds(sid * RPT + k * B, B)])
        plsc.subcore_barrier()

        def start_gather(j, b):
            off = jnp.minimum(j, BPS - 1) * B
            pltpu.async_copy(x_hbm.at[src_v.at[pl.ds(off, B)]], rows[b],
                             gs[b])

        def wait_gather(b):
            pltpu.make_async_copy(x_hbm.at[pl.ds(0, B)], rows[b],
                                  gs[b]).wait()

        def start_scatter(j, b):
            pltpu.async_copy(rows[b], accum.at[dst_v.at[j]], ss[b],
                             add=True)

        def wait_scatter(b):
            pltpu.make_async_copy(rows[b], accum.at[pl.ds(0, B)],
                                  ss[b]).wait()

        def scale(j, b):
            @pl.loop(0, B, unroll=4)
            def _(i):
                wb = plsc.load_gather(
                    w_v, [jnp.full((LANES,), j * B + i, jnp.int32)])
                for c in range(D // LANES):
                    rows[b].at[i, pl.ds(c * LANES, LANES)][...] = (
                        rows[b].at[i, pl.ds(c * LANES, LANES)][...] * wb)

        @pl.loop(0, NSTG)
        def _(s):
            pltpu.sync_copy(src_hbm.at[pl.ds(base + s * SEG, SEG)], src_v)
            pltpu.sync_copy(dst_hbm.at[wid, s], dst_v)
            pltpu.sync_copy(w_hbm.at[pl.ds(base + s * SEG, SEG)], w_v)

            for b in range(NBUF):
                start_gather(b, b)

            @pl.loop(0, GROUPS)
            def _(k):
                j = NBUF * k
                for b in range(NBUF):
                    wait_gather(b)
                    scale(j + b, b)
                    start_scatter(j + b, b)
                    if b >= 1:
                        wait_scatter(b - 1)
                        start_gather(j + NBUF + b - 1, b - 1)
                wait_scatter(NBUF - 1)
                start_gather(j + 2 * NBUF - 1, NBUF - 1)

            for t in range(TAIL):
                wait_gather(t)
                scale(GROUPS * NBUF + t, t)
                start_scatter(GROUPS * NBUF + t, t)
            for t in range(TAIL):
                wait_scatter(t)
            for b in range(TAIL, NBUF):
                wait_gather(b)

        plsc.subcore_barrier()

        pltpu.sync_copy(accum.at[pl.ds(sid * RPT, RPT)],
                        out_hbm.at[cid, pl.ds(sid * RPT, RPT)])

    return spmm(x, src, dst_blocks, w)


def _tc_matmul_relu(p0, p1, W):
    BLK = 1000

    def mm(p0_ref, p1_ref, w_ref, o_ref):
        acc = p0_ref[...] + p1_ref[...]
        o_ref[...] = jnp.maximum(
            jnp.dot(acc, w_ref[...], preferred_element_type=jnp.float32), 0.0)

    return pl.pallas_call(
        mm,
        grid=(N_NODES // BLK,),
        in_specs=[
            pl.BlockSpec((BLK, D), lambda i: (i, 0)),
            pl.BlockSpec((BLK, D), lambda i: (i, 0)),
            pl.BlockSpec((D, D), lambda i: (0, 0)),
        ],
        out_specs=pl.BlockSpec((BLK, D), lambda i: (i, 0)),
        out_shape=jax.ShapeDtypeStruct((N_NODES, D), jnp.float32),
    )(p0, p1, W)


@jax.jit
def kernel(x, edge_index, edge_weight, W):
    src = edge_index[0]
    dst_blocks = edge_index[1].reshape(NW, NSTG, BPS, B)
    partials = _sc_spmm(x, src, dst_blocks, edge_weight)
    return _tc_matmul_relu(partials[0, :N_NODES], partials[1, :N_NODES],
                           W)


# speedup vs baseline: 11.1232x; 1.0390x over previous
import dataclasses
import functools

import jax
import jax.numpy as jnp
from jax import lax
from jax.experimental import pallas as pl
from jax.experimental.pallas import tpu as pltpu
from jax.experimental.pallas import tpu_sc as plsc

N_NODES = 10000
D = 128
LANES = 16
NC = 2
NS = 16
NW = NC * NS
E_TOTAL = 320000
EPW = E_TOTAL // NW
B = 40
SEG = 2000
NSTG = EPW // SEG
BPS = SEG // B
NBUF = 6
GROUPS = (BPS - 2) // NBUF
TAIL = BPS - GROUPS * NBUF
N_PAD = 10240
RPT = N_PAD // NS


def _sc_spmm(x, src, dst_blocks, w):
    mesh = plsc.VectorSubcoreMesh(core_axis_name="c", subcore_axis_name="s")

    cp = pltpu.CompilerParams()
    if "needs_layout_passes" in pltpu.CompilerParams.__dataclass_fields__:
        cp = dataclasses.replace(cp, needs_layout_passes=False)

    @functools.partial(
        pl.kernel,
        compiler_params=cp,
        out_type=jax.ShapeDtypeStruct((NC, N_PAD, D), jnp.float32),
        mesh=mesh,
        scratch_types=[
            pltpu.VMEM((SEG,), jnp.int32),
            pltpu.VMEM((BPS, B), jnp.int32),
            pltpu.VMEM((SEG,), jnp.float32),
            *[pltpu.VMEM((B, D), jnp.float32)] * NBUF,
            pltpu.VMEM_SHARED((N_PAD, D), jnp.float32),
            *[pltpu.SemaphoreType.DMA] * NBUF,
            *[pltpu.SemaphoreType.DMA] * NBUF,
        ],
    )
    def spmm(x_hbm, src_hbm, dst_hbm, w_hbm, out_hbm,
             src_v, dst_v, w_v, r0, r1, r2, r3, r4, r5, accum,
             g0, g1, g2, g3, g4, g5, s0, s1, s2, s3, s4, s5):
        rows = [r0, r1, r2, r3, r4, r5]
        gs = [g0, g1, g2, g3, g4, g5]
        ss = [s0, s1, s2, s3, s4, s5]
        cid = lax.axis_index("c")
        sid = lax.axis_index("s")
        wid = cid * NS + sid
        base = wid * EPW

        @pl.loop(0, B)
        def _(r):
            for c in range(D // LANES):
                rows[0].at[r, pl.ds(c * LANES, LANES)][...] = jnp.zeros(
                    (LANES,), jnp.float32)

        for k in range(RPT // B):
            pltpu.sync_copy(rows[0], accum.at[pl.ds(sid * RPT + k * B, B)])
        plsc.subcore_barrier()

        def start_gather(j, b):
            off = jnp.minimum(j, BPS - 1) * B
            pltpu.async_copy(x_hbm.at[src_v.at[pl.ds(off, B)]], rows[b],
                             gs[b])

        def wait_gather(b):
            pltpu.make_async_copy(x_hbm.at[pl.ds(0, B)], rows[b],
                                  gs[b]).wait()

        def start_scatter(j, b):
            pltpu.async_copy(rows[b], accum.at[dst_v.at[j]], ss[b],
                             add=True)

        def wait_scatter(b):
            pltpu.make_async_copy(rows[b], accum.at[pl.ds(0, B)],
                                  ss[b]).wait()

        def scale(j, b):
            @plsc.parallel_loop(0, B, unroll=8)
            def _(i):
                wb = plsc.load_gather(
                    w_v, [jnp.full((LANES,), j * B + i, jnp.int32)])
                for c in range(D // LANES):
                    rows[b].at[i, pl.ds(c * LANES, LANES)][...] = (
                        rows[b].at[i, pl.ds(c * LANES, LANES)][...] * wb)

        @pl.loop(0, NSTG)
        def _(s):
            pltpu.sync_copy(src_hbm.at[pl.ds(base + s * SEG, SEG)], src_v)
            pltpu.sync_copy(dst_hbm.at[wid, s], dst_v)
            pltpu.sync_copy(w_hbm.at[pl.ds(base + s * SEG, SEG)], w_v)

            for b in range(NBUF):
                start_gather(b, b)

            @pl.loop(0, GROUPS)
            def _(k):
                j = NBUF * k
                for b in range(NBUF):
                    wait_gather(b)
                    scale(j + b, b)
                    start_scatter(j + b, b)
                    if b >= 1:
                        wait_scatter(b - 1)
                        start_gather(j + NBUF + b - 1, b - 1)
                wait_scatter(NBUF - 1)
                start_gather(j + 2 * NBUF - 1, NBUF - 1)

            for t in range(TAIL):
                wait_gather(t)
                scale(GROUPS * NBUF + t, t)
                start_scatter(GROUPS * NBUF + t, t)
            for t in range(TAIL):
                wait_scatter(t)
            for b in range(TAIL, NBUF):
                wait_gather(b)

        plsc.subcore_barrier()

        pltpu.sync_copy(accum.at[pl.ds(sid * RPT, RPT)],
                        out_hbm.at[cid, pl.ds(sid * RPT, RPT)])

    return spmm(x, src, dst_blocks, w)


def _tc_matmul_relu(p0, p1, W):
    BLK = 1000

    def mm(p0_ref, p1_ref, w_ref, o_ref):
        acc = p0_ref[...] + p1_ref[...]
        o_ref[...] = jnp.maximum(
            jnp.dot(acc, w_ref[...], preferred_element_type=jnp.float32), 0.0)

    return pl.pallas_call(
        mm,
        grid=(N_NODES // BLK,),
        in_specs=[
            pl.BlockSpec((BLK, D), lambda i: (i, 0)),
            pl.BlockSpec((BLK, D), lambda i: (i, 0)),
            pl.BlockSpec((D, D), lambda i: (0, 0)),
        ],
        out_specs=pl.BlockSpec((BLK, D), lambda i: (i, 0)),
        out_shape=jax.ShapeDtypeStruct((N_NODES, D), jnp.float32),
    )(p0, p1, W)


@jax.jit
def kernel(x, edge_index, edge_weight, W):
    src = edge_index[0]
    dst_blocks = edge_index[1].reshape(NW, NSTG, BPS, B)
    partials = _sc_spmm(x, src, dst_blocks, edge_weight)
    return _tc_matmul_relu(partials[0, :N_NODES], partials[1, :N_NODES],
                           W)


# speedup vs baseline: 11.2312x; 1.0097x over previous
import dataclasses
import functools

import jax
import jax.numpy as jnp
from jax import lax
from jax.experimental import pallas as pl
from jax.experimental.pallas import tpu as pltpu
from jax.experimental.pallas import tpu_sc as plsc

N_NODES = 10000
D = 128
LANES = 16
NC = 2
NS = 16
NW = NC * NS
E_TOTAL = 320000
EPW = E_TOTAL // NW
B = 80
SEG = 2000
NSTG = EPW // SEG
BPS = SEG // B
NBUF = 3
GROUPS = BPS // NBUF
TAIL = BPS % NBUF
N_PAD = 10240
RPT = N_PAD // NS


def _sc_spmm(x, src, dst_blocks, w):
    mesh = plsc.VectorSubcoreMesh(core_axis_name="c", subcore_axis_name="s")

    cp = pltpu.CompilerParams()
    if "needs_layout_passes" in pltpu.CompilerParams.__dataclass_fields__:
        cp = dataclasses.replace(cp, needs_layout_passes=False)

    @functools.partial(
        pl.kernel,
        compiler_params=cp,
        out_type=jax.ShapeDtypeStruct((NC, N_PAD, D), jnp.float32),
        mesh=mesh,
        scratch_types=[
            pltpu.VMEM((SEG,), jnp.int32),
            pltpu.VMEM((BPS, B), jnp.int32),
            pltpu.VMEM((SEG,), jnp.float32),
            *[pltpu.VMEM((B, D), jnp.float32)] * NBUF,
            pltpu.VMEM_SHARED((N_PAD, D), jnp.float32),
            *[pltpu.SemaphoreType.DMA] * NBUF,
            *[pltpu.SemaphoreType.DMA] * NBUF,
        ],
    )
    def spmm(x_hbm, src_hbm, dst_hbm, w_hbm, out_hbm,
             src_v, dst_v, w_v, r0, r1, r2, accum,
             g0, g1, g2, s0, s1, s2):
        rows = [r0, r1, r2]
        gs = [g0, g1, g2]
        ss = [s0, s1, s2]
        cid = lax.axis_index("c")
        sid = lax.axis_index("s")
        wid = cid * NS + sid
        base = wid * EPW

        @pl.loop(0, B)
        def _(r):
            for c in range(D // LANES):
                rows[0].at[r, pl.ds(c * LANES, LANES)][...] = jnp.zeros(
                    (LANES,), jnp.float32)

        for k in range(RPT // B):
            pltpu.sync_copy(rows[0], accum.at[pl.ds(sid * RPT + k * B, B)])
        plsc.subcore_barrier()

        def start_gather(j, b):
            off = jnp.minimum(j, BPS - 1) * B
            pltpu.async_copy(x_hbm.at[src_v.at[pl.ds(off, B)]], rows[b],
                             gs[b])

        def wait_gather(b):
            pltpu.make_async_copy(x_hbm.at[pl.ds(0, B)], rows[b],
                                  gs[b]).wait()

        def start_scatter(j, b):
            pltpu.async_copy(rows[b], accum.at[dst_v.at[j]], ss[b],
                             add=True)

        def wait_scatter(b):
            pltpu.make_async_copy(rows[b], accum.at[pl.ds(0, B)],
                                  ss[b]).wait()

        def scale(j, b):
            @plsc.parallel_loop(0, B, unroll=8)
            def _(i):
                wb = plsc.load_gather(
                    w_v, [jnp.full((LANES,), j * B + i, jnp.int32)])
                for c in range(D // LANES):
                    rows[b].at[i, pl.ds(c * LANES, LANES)][...] = (
                        rows[b].at[i, pl.ds(c * LANES, LANES)][...] * wb)

        @pl.loop(0, NSTG)
        def _(s):
            pltpu.sync_copy(src_hbm.at[pl.ds(base + s * SEG, SEG)], src_v)
            pltpu.sync_copy(dst_hbm.at[wid, s], dst_v)
            pltpu.sync_copy(w_hbm.at[pl.ds(base + s * SEG, SEG)], w_v)

            for b in range(NBUF):
                start_gather(b, b)

            @pl.loop(0, GROUPS)
            def _(k):
                j = NBUF * k
                for b in range(NBUF):
                    wait_gather(b)
                    scale(j + b, b)
                    start_scatter(j + b, b)
                    if b >= 1:
                        wait_scatter(b - 1)
                        start_gather(j + NBUF + b - 1, b - 1)
                wait_scatter(NBUF - 1)
                start_gather(j + 2 * NBUF - 1, NBUF - 1)

            for t in range(TAIL):
                wait_gather(t)
                scale(GROUPS * NBUF + t, t)
                start_scatter(GROUPS * NBUF + t, t)
            for t in range(TAIL):
                wait_scatter(t)
            for b in range(TAIL, NBUF):
                wait_gather(b)

        plsc.subcore_barrier()

        pltpu.sync_copy(accum.at[pl.ds(sid * RPT, RPT)],
                        out_hbm.at[cid, pl.ds(sid * RPT, RPT)])

    return spmm(x, src, dst_blocks, w)


def _tc_matmul_relu(partials, W):
    BLK = 2000

    def mm(p_ref, w_ref, o_ref):
        acc = p_ref[0] + p_ref[1]
        o_ref[...] = jnp.maximum(
            jnp.dot(acc, w_ref[...], preferred_element_type=jnp.float32), 0.0)

    return pl.pallas_call(
        mm,
        grid=(N_NODES // BLK,),
        in_specs=[
            pl.BlockSpec((2, BLK, D), lambda i: (0, i, 0)),
            pl.BlockSpec((D, D), lambda i: (0, 0)),
        ],
        out_specs=pl.BlockSpec((BLK, D), lambda i: (i, 0)),
        out_shape=jax.ShapeDtypeStruct((N_NODES, D), jnp.float32),
    )(partials, W)


@jax.jit
def kernel(x, edge_index, edge_weight, W):
    src = edge_index[0]
    dst_blocks = edge_index[1].reshape(NW, NSTG, BPS, B)
    partials = _sc_spmm(x, src, dst_blocks, edge_weight)
    return _tc_matmul_relu(partials, W)


# speedup vs baseline: 11.8881x; 1.0585x over previous
import dataclasses
import functools

import jax
import jax.numpy as jnp
from jax import lax
from jax.experimental import pallas as pl
from jax.experimental.pallas import tpu as pltpu
from jax.experimental.pallas import tpu_sc as plsc

N_NODES = 10000
D = 128
LANES = 16
NC = 2
NS = 16
NW = NC * NS
E_TOTAL = 320000
EPW = E_TOTAL // NW
B = 40
SEG = 2000
NSTG = EPW // SEG
BPS = SEG // B
NBUF = 4
GROUPS = BPS // NBUF
TAIL = BPS % NBUF
N_PAD = 10240
RPT = N_PAD // NS


def _sc_spmm(x, src, dst_blocks, w):
    mesh = plsc.VectorSubcoreMesh(core_axis_name="c", subcore_axis_name="s")

    cp = pltpu.CompilerParams()
    if "needs_layout_passes" in pltpu.CompilerParams.__dataclass_fields__:
        cp = dataclasses.replace(cp, needs_layout_passes=False)

    @functools.partial(
        pl.kernel,
        compiler_params=cp,
        out_type=jax.ShapeDtypeStruct((NC, N_PAD, D), jnp.float32),
        mesh=mesh,
        scratch_types=[
            *[pltpu.VMEM((SEG,), jnp.int32)] * 2,
            *[pltpu.VMEM((BPS, B), jnp.int32)] * 2,
            *[pltpu.VMEM((SEG,), jnp.float32)] * 2,
            *[pltpu.VMEM((B, D), jnp.float32)] * NBUF,
            pltpu.VMEM_SHARED((N_PAD, D), jnp.float32),
            *[pltpu.SemaphoreType.DMA] * NBUF,
            *[pltpu.SemaphoreType.DMA] * NBUF,
            *[pltpu.SemaphoreType.DMA] * 2,
        ],
    )
    def spmm(x_hbm, src_hbm, dst_hbm, w_hbm, out_hbm,
             sv0, sv1, dv0, dv1, wv0, wv1, r0, r1, r2, r3, accum,
             g0, g1, g2, g3, s0, s1, s2, s3, st0, st1):
        rows = [r0, r1, r2, r3]
        gs = [g0, g1, g2, g3]
        ss = [s0, s1, s2, s3]
        srcv, dstv, wv, st = [sv0, sv1], [dv0, dv1], [wv0, wv1], [st0, st1]
        cid = lax.axis_index("c")
        sid = lax.axis_index("s")
        wid = cid * NS + sid
        base = wid * EPW

        def start_stage(s, m):
            pltpu.async_copy(src_hbm.at[pl.ds(base + s * SEG, SEG)],
                             srcv[m], st[m])
            pltpu.async_copy(dst_hbm.at[wid, s], dstv[m], st[m])
            pltpu.async_copy(w_hbm.at[pl.ds(base + s * SEG, SEG)],
                             wv[m], st[m])

        def wait_stage(m):
            pltpu.make_async_copy(src_hbm.at[pl.ds(base, SEG)],
                                  srcv[m], st[m]).wait()
            pltpu.make_async_copy(dst_hbm.at[wid], dstv[m], st[m]).wait()
            pltpu.make_async_copy(w_hbm.at[pl.ds(base, SEG)],
                                  wv[m], st[m]).wait()

        start_stage(0, 0)

        @pl.loop(0, B)
        def _(r):
            for c in range(D // LANES):
                rows[0].at[r, pl.ds(c * LANES, LANES)][...] = jnp.zeros(
                    (LANES,), jnp.float32)

        for k in range(RPT // B):
            pltpu.sync_copy(rows[0], accum.at[pl.ds(sid * RPT + k * B, B)])
        plsc.subcore_barrier()

        def start_gather(j, b, m):
            off = jnp.minimum(j, BPS - 1) * B
            pltpu.async_copy(x_hbm.at[srcv[m].at[pl.ds(off, B)]], rows[b],
                             gs[b])

        def wait_gather(b):
            pltpu.make_async_copy(x_hbm.at[pl.ds(0, B)], rows[b],
                                  gs[b]).wait()

        def start_scatter(j, b, m):
            pltpu.async_copy(rows[b], accum.at[dstv[m].at[j]], ss[b],
                             add=True)

        def wait_scatter(b):
            pltpu.make_async_copy(rows[b], accum.at[pl.ds(0, B)],
                                  ss[b]).wait()

        def scale(j, b, m):
            @plsc.parallel_loop(0, B, unroll=8)
            def _(i):
                wb = plsc.load_gather(
                    wv[m], [jnp.full((LANES,), j * B + i, jnp.int32)])
                for c in range(D // LANES):
                    rows[b].at[i, pl.ds(c * LANES, LANES)][...] = (
                        rows[b].at[i, pl.ds(c * LANES, LANES)][...] * wb)

        for s in range(NSTG):
            m = s & 1
            wait_stage(m)
            if s + 1 < NSTG:
                start_stage(s + 1, (s + 1) & 1)

            for b in range(NBUF):
                start_gather(b, b, m)

            @pl.loop(0, GROUPS)
            def _(k):
                j = NBUF * k
                for b in range(NBUF):
                    wait_gather(b)
                    scale(j + b, b, m)
                    start_scatter(j + b, b, m)
                    if b >= 1:
                        wait_scatter(b - 1)
                        start_gather(j + NBUF + b - 1, b - 1, m)
                wait_scatter(NBUF - 1)
                start_gather(j + 2 * NBUF - 1, NBUF - 1, m)

            for t in range(TAIL):
                wait_gather(t)
                scale(GROUPS * NBUF + t, t, m)
                start_scatter(GROUPS * NBUF + t, t, m)
            for t in range(TAIL):
                wait_scatter(t)
            for b in range(TAIL, NBUF):
                wait_gather(b)

        plsc.subcore_barrier()

        pltpu.sync_copy(accum.at[pl.ds(sid * RPT, RPT)],
                        out_hbm.at[cid, pl.ds(sid * RPT, RPT)])

    return spmm(x, src, dst_blocks, w)


def _tc_matmul_relu(partials, W):
    BLK = 2000

    def mm(p_ref, w_ref, o_ref):
        acc = p_ref[0] + p_ref[1]
        o_ref[...] = jnp.maximum(
            jnp.dot(acc, w_ref[...], preferred_element_type=jnp.float32), 0.0)

    return pl.pallas_call(
        mm,
        grid=(N_NODES // BLK,),
        in_specs=[
            pl.BlockSpec((2, BLK, D), lambda i: (0, i, 0)),
            pl.BlockSpec((D, D), lambda i: (0, 0)),
        ],
        out_specs=pl.BlockSpec((BLK, D), lambda i: (i, 0)),
        out_shape=jax.ShapeDtypeStruct((N_NODES, D), jnp.float32),
    )(partials, W)


@jax.jit
def kernel(x, edge_index, edge_weight, W):
    src = edge_index[0]
    dst_blocks = edge_index[1].reshape(NW, NSTG, BPS, B)
    partials = _sc_spmm(x, src, dst_blocks, edge_weight)
    return _tc_matmul_relu(partials, W)
